# baseline (speedup 1.0000x reference)
import functools

import jax
import jax.numpy as jnp
from jax import lax
from jax.experimental import pallas as pl
from jax.experimental.pallas import tpu as pltpu
from jax.experimental.pallas import tpu_sc as plsc

_NC = 2
_NS = 16
_LANES = 16


def _round_bf16_bits(f):
    u = jax.lax.bitcast_convert_type(f, jnp.uint32)
    return (u + jnp.uint32(0x7FFF) + ((u >> 16) & jnp.uint32(1))) >> 16


def _eproj_body(a_ref, w_ref, b_ref, o_ref):
    e = (
        jnp.dot(a_ref[...], w_ref[...], preferred_element_type=jnp.float32)
        + b_ref[...]
    )
    D = e.shape[1]
    lo = _round_bf16_bits(e[:, : D // 2])
    hi = _round_bf16_bits(e[:, D // 2:])
    o_ref[...] = jax.lax.bitcast_convert_type(lo | (hi << 16), jnp.int32)


def _edge_projection(edge_attr, We, be):
    E, DE = edge_attr.shape
    D = We.shape[1]
    BE = 12800
    assert E % BE == 0
    return pl.pallas_call(
        _eproj_body,
        grid=(E // BE,),
        in_specs=[
            pl.BlockSpec((BE, DE), lambda i: (i, 0)),
            pl.BlockSpec((DE, D), lambda i: (0, 0)),
            pl.BlockSpec((1, D), lambda i: (0, 0)),
        ],
        out_specs=pl.BlockSpec((BE, D // 2), lambda i: (i, 0)),
        out_shape=jax.ShapeDtypeStruct((E, D // 2), jnp.int32),
    )(edge_attr, We, be.reshape(1, D))


_NB = 3


def _sc_edge_body(N, D, E, C, x_hbm, src_hbm, dst_hbm, e_hbm, out_hbm, *sc):
    src_v = sc[0:_NB]
    dst_v = sc[_NB:2 * _NB]
    ebuf = sc[2 * _NB]
    xbuf = sc[2 * _NB + 1]
    aggr_sh = sc[2 * _NB + 2]
    sem_i = sc[2 * _NB + 3:2 * _NB + 3 + _NB]
    sem_e = sc[2 * _NB + 3 + _NB:2 * _NB + 3 + 2 * _NB]
    sem_g = sc[2 * _NB + 3 + 2 * _NB:2 * _NB + 3 + 3 * _NB]
    sem_s = sc[2 * _NB + 3 + 3 * _NB:2 * _NB + 3 + 4 * _NB]
    cid = lax.axis_index("c")
    sid = lax.axis_index("s")
    wid = cid * _NS + sid

    epw = E // (_NC * _NS)
    nchunk = epw // C
    zrows = C
    nrow_chunks = N // zrows

    zvec = jnp.zeros((_LANES,), jnp.float32)

    def _zero_row(i, _):
        for j in range(D // _LANES):
            xbuf[i, pl.ds(j * _LANES, _LANES)] = zvec
        return 0

    lax.fori_loop(0, zrows, _zero_row, 0)
    zslice = xbuf.at[pl.ds(0, zrows)]
    for k in range((nrow_chunks + _NS - 1) // _NS):
        c = sid + k * _NS

        @pl.when(c < nrow_chunks)
        def _():
            pltpu.sync_copy(zslice, aggr_sh.at[pl.ds(pl.multiple_of(c * zrows, 8), zrows)])

    plsc.subcore_barrier()

    def _base(j):
        return pl.multiple_of(wid * epw + j * C, 8)

    def issue_idx(j, b):
        pltpu.async_copy(src_hbm.at[pl.ds(_base(j), C)], src_v[b], sem_i[b])
        pltpu.async_copy(dst_hbm.at[pl.ds(_base(j), C)], dst_v[b], sem_i[b])

    def wait_idx(j, b):
        pltpu.make_async_copy(src_hbm.at[pl.ds(_base(j), C)], src_v[b], sem_i[b]).wait()
        pltpu.make_async_copy(dst_hbm.at[pl.ds(_base(j), C)], dst_v[b], sem_i[b]).wait()

    def _ebs(b):
        return ebuf.at[pl.ds(b * C, C)]

    def _xbs(b):
        return xbuf.at[pl.ds(b * C, C)]

    def issue_e(j, b):
        pltpu.async_copy(e_hbm.at[pl.ds(_base(j), C)], _ebs(b), sem_e[b])

    def wait_e(j, b):
        pltpu.make_async_copy(e_hbm.at[pl.ds(_base(j), C)], _ebs(b), sem_e[b]).wait()

    def issue_g(b):
        pltpu.async_copy(x_hbm.at[src_v[b]], _xbs(b), sem_g[b])

    def wait_g(b):
        pltpu.make_async_copy(x_hbm.at[src_v[b]], _xbs(b), sem_g[b]).wait()

    def issue_s(b):
        pltpu.async_copy(_xbs(b), aggr_sh.at[dst_v[b]], sem_s[b], add=True)

    def wait_s(b):
        pltpu.make_async_copy(_xbs(b), aggr_sh.at[dst_v[b]], sem_s[b]).wait()

    def compute(b):
        hi16 = jnp.int32(16)

        @plsc.parallel_loop(0, C, step=1, unroll=2)
        def _row(i):
            r = b * C + i
            for g in range(D // (2 * _LANES)):
                w = ebuf[r, pl.ds(g * _LANES, _LANES)]
                elo = jax.lax.bitcast_convert_type(w << hi16, jnp.float32)
                ehi = jax.lax.bitcast_convert_type(
                    (w >> hi16) << hi16, jnp.float32)
                sl0 = pl.ds(g * _LANES, _LANES)
                sl1 = pl.ds(D // 2 + g * _LANES, _LANES)
                xbuf[r, sl0] = jnp.maximum(xbuf[r, sl0] + elo, 0.0)
                xbuf[r, sl1] = jnp.maximum(xbuf[r, sl1] + ehi, 0.0)

    def _step(j, b):
        wait_g(b)
        wait_e(j, b)
        b1 = (b + 1) % _NB
        b2 = (b + 2) % _NB

        @pl.when(j + 1 < nchunk)
        def _():
            wait_idx(j + 1, b1)
            issue_g(b1)

        compute(b)
        issue_s(b)

        @pl.when(j > 0)
        def _():
            wait_s(b2)

        @pl.when(j + 2 < nchunk)
        def _():
            issue_e(j + 2, b2)
            issue_idx(j + 2, b2)

    assert nchunk >= 2 * _NB
    ntrip = nchunk // _NB
    ntail = nchunk - _NB * ntrip

    for b in range(_NB - 1):
        issue_idx(b, b)
        issue_e(b, b)
    wait_idx(0, 0)
    issue_g(0)

    def _trip(t, _):
        for k in range(_NB):
            _step(_NB * t + k, k)
        return 0

    lax.fori_loop(0, ntrip, _trip, 0)
    for r in range(ntail):
        _step(_NB * ntrip + r, r)
    wait_s((nchunk - 1) % _NB)
    plsc.subcore_barrier()

    for k in range((nrow_chunks + _NS - 1) // _NS):
        c = sid + k * _NS

        @pl.when(c < nrow_chunks)
        def _():
            base = pl.multiple_of(c * zrows, 8)
            pltpu.sync_copy(aggr_sh.at[pl.ds(base, zrows)], zslice)
            pltpu.sync_copy(zslice, out_hbm.at[cid, pl.ds(base, zrows)])


def _sc_aggregate(x, edge_index, e, C):
    N, D = x.shape
    E = edge_index.shape[1]
    assert E % (_NC * _NS) == 0 and (E // (_NC * _NS)) % C == 0
    assert N % C == 0
    mesh = plsc.VectorSubcoreMesh(core_axis_name="c", subcore_axis_name="s")
    kern = functools.partial(
        pl.kernel,
        mesh=mesh,
        out_type=jax.ShapeDtypeStruct((_NC, N, D), jnp.float32),
        scratch_types=(
            [pltpu.VMEM((C,), jnp.int32) for _ in range(2 * _NB)]
            + [
                pltpu.VMEM((_NB * C, D // 2), jnp.int32),
                pltpu.VMEM((_NB * C, D), jnp.float32),
                pltpu.VMEM_SHARED((N, D), jnp.float32),
            ]
            + [pltpu.SemaphoreType.DMA for _ in range(4 * _NB)]
        ),
    )(functools.partial(_sc_edge_body, N, D, E, C))
    return kern(x, edge_index[0], edge_index[1], e)


def _mlp_body(x_ref, p_ref, w1_ref, b1_ref, w2_ref, b2_ref, o_ref):
    h = x_ref[...] + p_ref[0] + p_ref[1]
    h = jnp.maximum(
        jnp.dot(h, w1_ref[...], preferred_element_type=jnp.float32) + b1_ref[...],
        0.0,
    )
    o_ref[...] = (
        jnp.dot(h, w2_ref[...], preferred_element_type=jnp.float32) + b2_ref[...]
    )


def _node_mlp(x, partials, W1, b1, W2, b2):
    N, D = x.shape
    BN = 2000
    assert N % BN == 0
    return pl.pallas_call(
        _mlp_body,
        grid=(N // BN,),
        in_specs=[
            pl.BlockSpec((BN, D), lambda i: (i, 0)),
            pl.BlockSpec((_NC, BN, D), lambda i: (0, i, 0)),
            pl.BlockSpec((D, D), lambda i: (0, 0)),
            pl.BlockSpec((1, D), lambda i: (0, 0)),
            pl.BlockSpec((D, D), lambda i: (0, 0)),
            pl.BlockSpec((1, D), lambda i: (0, 0)),
        ],
        out_specs=pl.BlockSpec((BN, D), lambda i: (i, 0)),
        out_shape=jax.ShapeDtypeStruct((N, D), jnp.float32),
    )(x, partials, W1, b1.reshape(1, D), W2, b2.reshape(1, D))


def kernel(x, edge_index, edge_attr, We, be, W1, b1, W2, b2):
    e = _edge_projection(edge_attr, We, be)
    partials = _sc_aggregate(x, edge_index, e, C=40)
    return _node_mlp(x, partials, W1, b1, W2, b2)


# speedup vs baseline: 1.1321x; 1.1321x over previous
import functools

import jax
import jax.numpy as jnp
from jax import lax
from jax.experimental import pallas as pl
from jax.experimental.pallas import tpu as pltpu
from jax.experimental.pallas import tpu_sc as plsc

_NC = 2
_NS = 16
_LANES = 16


def _eproj_body(a_ref, w_ref, b_ref, o_ref):
    o_ref[...] = (
        jnp.dot(a_ref[...], w_ref[...], preferred_element_type=jnp.float32)
        + b_ref[...]
    )


def _edge_projection(edge_attr, We, be):
    E, DE = edge_attr.shape
    D = We.shape[1]
    BE = 12800
    assert E % BE == 0
    return pl.pallas_call(
        _eproj_body,
        grid=(E // BE,),
        in_specs=[
            pl.BlockSpec((BE, DE), lambda i: (i, 0)),
            pl.BlockSpec((DE, D), lambda i: (0, 0)),
            pl.BlockSpec((1, D), lambda i: (0, 0)),
        ],
        out_specs=pl.BlockSpec((BE, D), lambda i: (i, 0)),
        out_shape=jax.ShapeDtypeStruct((E, D), jnp.float32),
    )(edge_attr, We, be.reshape(1, D))


_NB = 4


def _sc_edge_body(N, D, E, C, x_hbm, src_hbm, dst_hbm, e_hbm, out_hbm, *sc):
    src_v = sc[0:_NB]
    dst_v = sc[_NB:2 * _NB]
    ebuf = sc[2 * _NB:3 * _NB]
    zbuf = sc[3 * _NB]
    aggr_sh = sc[3 * _NB + 1]
    sem_i = sc[3 * _NB + 2:3 * _NB + 2 + _NB]
    sem_e = sc[3 * _NB + 2 + _NB:3 * _NB + 2 + 2 * _NB]
    sem_g = sc[3 * _NB + 2 + 2 * _NB:3 * _NB + 2 + 3 * _NB]
    sem_s = sc[3 * _NB + 2 + 3 * _NB:3 * _NB + 2 + 4 * _NB]
    cid = lax.axis_index("c")
    sid = lax.axis_index("s")
    wid = cid * _NS + sid

    epw = E // (_NC * _NS)
    nchunk = epw // C
    zrows = zbuf.shape[0]
    nrow_chunks = N // zrows

    zvec = jnp.zeros((_LANES,), jnp.float32)

    def _zero_row(i, _):
        for j in range(D // _LANES):
            zbuf[i, pl.ds(j * _LANES, _LANES)] = zvec
        return 0

    lax.fori_loop(0, zrows, _zero_row, 0)
    for k in range((nrow_chunks + _NS - 1) // _NS):
        c = sid + k * _NS

        @pl.when(c < nrow_chunks)
        def _():
            pltpu.sync_copy(zbuf, aggr_sh.at[pl.ds(pl.multiple_of(c * zrows, 8), zrows)])

    plsc.subcore_barrier()

    def _base(j):
        return pl.multiple_of(wid * epw + j * C, 8)

    def issue_idx(j, b):
        pltpu.async_copy(src_hbm.at[pl.ds(_base(j), C)], src_v[b], sem_i[b])
        pltpu.async_copy(dst_hbm.at[pl.ds(_base(j), C)], dst_v[b], sem_i[b])

    def wait_idx(j, b):
        pltpu.make_async_copy(src_hbm.at[pl.ds(_base(j), C)], src_v[b], sem_i[b]).wait()
        pltpu.make_async_copy(dst_hbm.at[pl.ds(_base(j), C)], dst_v[b], sem_i[b]).wait()

    def issue_e(j, b):
        pltpu.async_copy(e_hbm.at[pl.ds(_base(j), C)], ebuf[b], sem_e[b])

    def wait_e(j, b):
        pltpu.make_async_copy(e_hbm.at[pl.ds(_base(j), C)], ebuf[b], sem_e[b]).wait()

    def issue_g(b):
        pltpu.async_copy(x_hbm.at[src_v[b]], ebuf[b], sem_g[b], add=True)

    def wait_g(b):
        pltpu.make_async_copy(x_hbm.at[src_v[b]], ebuf[b], sem_g[b]).wait()

    def issue_s(b):
        pltpu.async_copy(ebuf[b], aggr_sh.at[dst_v[b]], sem_s[b], add=True)

    def wait_s(b):
        pltpu.make_async_copy(ebuf[b], aggr_sh.at[dst_v[b]], sem_s[b]).wait()

    def relu(b):
        eb = ebuf[b]

        @plsc.parallel_loop(0, C, step=1, unroll=4)
        def _row(i):
            for jj in range(D // _LANES):
                sl = pl.ds(jj * _LANES, _LANES)
                eb[i, sl] = jnp.maximum(eb[i, sl], 0.0)

    assert nchunk % _NB == 1 and nchunk >= _NB + 1
    nquad = nchunk // _NB

    for b in range(_NB - 1):
        issue_idx(b, b)
        issue_e(b, b)
    wait_idx(0, 0)
    wait_e(0, 0)
    issue_g(0)

    def _step(j, b):
        wait_g(b)
        b1 = (b + 1) % _NB
        wait_idx(j + 1, b1)
        wait_e(j + 1, b1)
        issue_g(b1)
        relu(b)
        issue_s(b)
        b3 = (b + _NB - 1) % _NB

        @pl.when(j > 0)
        def _():
            wait_s(b3)

        @pl.when(j + _NB - 1 < nchunk)
        def _():
            issue_e(j + _NB - 1, b3)
            issue_idx(j + _NB - 1, b3)

    def _quad(t, _):
        for k in range(_NB):
            _step(_NB * t + k, k)
        return 0

    lax.fori_loop(0, nquad, _quad, 0)
    wait_g(0)
    relu(0)
    issue_s(0)
    wait_s(_NB - 1)
    wait_s(0)
    plsc.subcore_barrier()

    for k in range((nrow_chunks + _NS - 1) // _NS):
        c = sid + k * _NS

        @pl.when(c < nrow_chunks)
        def _():
            base = pl.multiple_of(c * zrows, 8)
            pltpu.sync_copy(aggr_sh.at[pl.ds(base, zrows)], zbuf)
            pltpu.sync_copy(zbuf, out_hbm.at[cid, pl.ds(base, zrows)])


def _sc_aggregate(x, edge_index, e, C):
    N, D = x.shape
    E = edge_index.shape[1]
    assert E % (_NC * _NS) == 0 and (E // (_NC * _NS)) % C == 0
    zrows = 40
    assert N % zrows == 0
    mesh = plsc.VectorSubcoreMesh(core_axis_name="c", subcore_axis_name="s")
    kern = functools.partial(
        pl.kernel,
        mesh=mesh,
        out_type=jax.ShapeDtypeStruct((_NC, N, D), jnp.float32),
        scratch_types=(
            [pltpu.VMEM((C,), jnp.int32) for _ in range(2 * _NB)]
            + [pltpu.VMEM((C, D), jnp.float32) for _ in range(_NB)]
            + [
                pltpu.VMEM((zrows, D), jnp.float32),
                pltpu.VMEM_SHARED((N, D), jnp.float32),
            ]
            + [pltpu.SemaphoreType.DMA for _ in range(4 * _NB)]
        ),
    )(functools.partial(_sc_edge_body, N, D, E, C))
    return kern(x, edge_index[0], edge_index[1], e)


def _mlp_body(x_ref, p_ref, w1_ref, b1_ref, w2_ref, b2_ref, o_ref):
    h = x_ref[...] + p_ref[0] + p_ref[1]
    h = jnp.maximum(
        jnp.dot(h, w1_ref[...], preferred_element_type=jnp.float32) + b1_ref[...],
        0.0,
    )
    o_ref[...] = (
        jnp.dot(h, w2_ref[...], preferred_element_type=jnp.float32) + b2_ref[...]
    )


def _node_mlp(x, partials, W1, b1, W2, b2):
    N, D = x.shape
    BN = 2000
    assert N % BN == 0
    return pl.pallas_call(
        _mlp_body,
        grid=(N // BN,),
        in_specs=[
            pl.BlockSpec((BN, D), lambda i: (i, 0)),
            pl.BlockSpec((_NC, BN, D), lambda i: (0, i, 0)),
            pl.BlockSpec((D, D), lambda i: (0, 0)),
            pl.BlockSpec((1, D), lambda i: (0, 0)),
            pl.BlockSpec((D, D), lambda i: (0, 0)),
            pl.BlockSpec((1, D), lambda i: (0, 0)),
        ],
        out_specs=pl.BlockSpec((BN, D), lambda i: (i, 0)),
        out_shape=jax.ShapeDtypeStruct((N, D), jnp.float32),
    )(x, partials, W1, b1.reshape(1, D), W2, b2.reshape(1, D))


def kernel(x, edge_index, edge_attr, We, be, W1, b1, W2, b2):
    e = _edge_projection(edge_attr, We, be)
    partials = _sc_aggregate(x, edge_index, e, C=80)
    return _node_mlp(x, partials, W1, b1, W2, b2)
